# baseline (speedup 1.0000x reference)
import functools

import jax
import jax.numpy as jnp
from jax import lax
from jax.experimental import pallas as pl
from jax.experimental.pallas import tpu as pltpu
from jax.experimental.pallas import tpu_sc as plsc

N_NODES = 10000
N_EDGES = 320000
D_FEAT = 128
HIDDEN = 128
HEADS = 8
HEAD_DIM = 16
GCOLS = 144
TCOLS = 16

PREP_BLK = 1024
FINAL_BLK = 1000

NC = 2
NS = 16
CHUNK = 112
N_PAD = 10240
ROWS_PER_TILE = N_PAD // NS
NW = NC * NS
PER_TILE = 90
NPAIRS = PER_TILE // 2
NCHUNKS = PER_TILE * NW
E_PAD = NCHUNKS * CHUNK


def _prep_body(x_ref, w_ref, asrc_ref, adst_ref, g_ref, t_ref):
    x = x_ref[...]
    w = w_ref[...]
    a_s = asrc_ref[...]
    a_d = adst_ref[...]
    k = lax.broadcasted_iota(jnp.int32, (HEADS, HIDDEN), 0)
    m = lax.broadcasted_iota(jnp.int32, (HEADS, HIDDEN), 1)
    sel = (m // HEAD_DIM) == k
    Bs = jnp.where(sel, jnp.concatenate([a_s] * HEADS, axis=1), 0.0)
    Bd = jnp.where(sel, jnp.concatenate([a_d] * HEADS, axis=1), 0.0)
    h = jnp.dot(x, w, preferred_element_type=jnp.float32)
    dn = (((1,), (1,)), ((), ()))
    a_src = lax.dot_general(h, Bs, dn, preferred_element_type=jnp.float32)
    a_dst = lax.dot_general(h, Bd, dn, preferred_element_type=jnp.float32)
    zero8 = jnp.zeros_like(a_src)
    g_ref[...] = jnp.concatenate([h, a_src, zero8], axis=1)
    t_ref[...] = jnp.concatenate([a_dst, zero8], axis=1)


def _prep(x, w, attn_src, attn_dst):
    grid = N_PAD // PREP_BLK
    return pl.pallas_call(
        _prep_body,
        grid=(grid,),
        in_specs=[
            pl.BlockSpec((PREP_BLK, D_FEAT), lambda i: (i, 0)),
            pl.BlockSpec((D_FEAT, HIDDEN), lambda i: (0, 0)),
            pl.BlockSpec((HEADS, HEAD_DIM), lambda i: (0, 0)),
            pl.BlockSpec((HEADS, HEAD_DIM), lambda i: (0, 0)),
        ],
        out_specs=[
            pl.BlockSpec((PREP_BLK, GCOLS), lambda i: (i, 0)),
            pl.BlockSpec((PREP_BLK, TCOLS), lambda i: (i, 0)),
        ],
        out_shape=[
            jax.ShapeDtypeStruct((N_PAD, GCOLS), jnp.float32),
            jax.ShapeDtypeStruct((N_PAD, TCOLS), jnp.float32),
        ],
    )(x, w, attn_src, attn_dst)


def _edge_body(g_hbm, t_hbm, ei_hbm, out_hbm, sidx0, didx0, sidx1, didx1,
               gbuf0, gbuf1, tbuf0, tbuf1, acc,
               sem_g0, sem_t0, sem_g1, sem_t1):
    c = lax.axis_index("c")
    s = lax.axis_index("s")
    wid = c * NS + s
    lanes = lax.iota(jnp.int32, 16)

    zeros16 = jnp.zeros((16,), jnp.float32)

    def zrow(i, carry):
        for kk in range(GCOLS // 16):
            gbuf0[i, pl.ds(kk * 16, 16)] = zeros16
        return carry

    lax.fori_loop(0, CHUNK, zrow, 0)

    r0 = s * ROWS_PER_TILE
    nfull = ROWS_PER_TILE // CHUNK
    for j in range(nfull):
        pltpu.sync_copy(gbuf0, acc.at[pl.ds(r0 + j * CHUNK, CHUNK)])
    rem = ROWS_PER_TILE - nfull * CHUNK
    pltpu.sync_copy(gbuf0.at[pl.ds(0, rem)],
                    acc.at[pl.ds(r0 + nfull * CHUNK, rem)])
    plsc.subcore_barrier()

    sx = (sidx0, sidx1)
    dx = (didx0, didx1)
    gb = (gbuf0, gbuf1)
    tb = (tbuf0, tbuf1)
    sg = (sem_g0, sem_g1)
    st = (sem_t0, sem_t1)

    def load_idx(j, pi):
        base = (wid + j * NW) * CHUNK
        pltpu.sync_copy(ei_hbm.at[0, pl.ds(base, CHUNK)], sx[pi])
        pltpu.sync_copy(ei_hbm.at[1, pl.ds(base, CHUNK)], dx[pi])

    def issue(pi):
        pltpu.async_copy(g_hbm.at[sx[pi]], gb[pi], sg[pi])
        pltpu.async_copy(t_hbm.at[dx[pi]], tb[pi], st[pi])

    def drain(pi):
        pltpu.make_async_copy(g_hbm.at[sx[pi]], gb[pi], sg[pi]).wait()
        pltpu.make_async_copy(t_hbm.at[dx[pi]], tb[pi], st[pi]).wait()

    def process(pi):
        gbuf, tbuf = gb[pi], tb[pi]

        def wrow(b, carry2):
            a_s = gbuf[b, pl.ds(HIDDEN, 16)]
            a_d = tbuf[b, pl.ds(0, 16)]
            e = a_s + a_d
            e = jnp.where(e >= 0.0, e, 0.2 * e)
            sv = jnp.where(lanes < HEADS, jnp.exp(e), 0.0)
            gbuf[b, pl.ds(HIDDEN, 16)] = sv
            for k in range(HEADS):
                gbuf[b, pl.ds(k * 16, 16)] = gbuf[b, pl.ds(k * 16, 16)] * sv[k]
            return carry2

        lax.fori_loop(0, CHUNK, wrow, 0)
        pltpu.sync_copy(gbuf, acc.at[dx[pi]], add=True)

    load_idx(0, 0)
    issue(0)

    def pair(jj, carry):
        j0 = 2 * jj
        load_idx(j0 + 1, 1)
        issue(1)
        drain(0)
        process(0)

        @pl.when(jj < NPAIRS - 1)
        def _():
            load_idx(j0 + 2, 0)
            issue(0)

        drain(1)
        process(1)
        return carry

    lax.fori_loop(0, NPAIRS, pair, 0)
    plsc.subcore_barrier()

    pltpu.sync_copy(acc.at[pl.ds(r0, ROWS_PER_TILE)],
                    out_hbm.at[c, pl.ds(r0, ROWS_PER_TILE)])


def _edges(g, t, edge_index2):
    mesh = plsc.VectorSubcoreMesh(core_axis_name="c", subcore_axis_name="s",
                                  num_cores=NC, num_subcores=NS)
    fn = functools.partial(
        pl.kernel,
        out_type=jax.ShapeDtypeStruct((NC, N_PAD, GCOLS), jnp.float32),
        mesh=mesh,
        scratch_types=(
            [pltpu.VMEM((CHUNK,), jnp.int32) for _ in range(4)]
            + [pltpu.VMEM((CHUNK, GCOLS), jnp.float32) for _ in range(2)]
            + [pltpu.VMEM((CHUNK, TCOLS), jnp.float32) for _ in range(2)]
            + [pltpu.VMEM_SHARED((N_PAD, GCOLS), jnp.float32)]
            + [pltpu.SemaphoreType.DMA for _ in range(4)]
        ),
        compiler_params=pltpu.CompilerParams(use_tc_tiling_on_sc=False),
    )(_edge_body)
    return fn(g, t, edge_index2)


def _final_body(p_ref, wd_ref, bd_ref, o_ref):
    ssum = p_ref[0] + p_ref[1]
    numer = ssum[:, :HIDDEN]
    denom = ssum[:, HIDDEN:HIDDEN + HEADS]
    k = lax.broadcasted_iota(jnp.int32, (HEADS, HIDDEN), 0)
    j = lax.broadcasted_iota(jnp.int32, (HEADS, HIDDEN), 1)
    rep = jnp.where((j // HEAD_DIM) == k, 1.0, 0.0)
    dex = jnp.dot(denom, rep, preferred_element_type=jnp.float32)
    x = jnp.maximum(numer / (dex + 1e-9), 0.0)
    y = jnp.dot(x, wd_ref[...], preferred_element_type=jnp.float32) + bd_ref[0, 0]
    o_ref[...] = 1.0 / (1.0 + jnp.exp(-y))


def _final(p, wd, bd):
    grid = N_NODES // FINAL_BLK
    return pl.pallas_call(
        _final_body,
        grid=(grid,),
        in_specs=[
            pl.BlockSpec((NC, FINAL_BLK, GCOLS), lambda i: (0, i, 0)),
            pl.BlockSpec((HIDDEN, 1), lambda i: (0, 0)),
            pl.BlockSpec((1, 1), lambda i: (0, 0)),
        ],
        out_specs=pl.BlockSpec((FINAL_BLK, 1), lambda i: (i, 0)),
        out_shape=jax.ShapeDtypeStruct((N_NODES, 1), jnp.float32),
    )(p, wd, bd)


@jax.jit
def kernel(node_features, edge_index, W, attn_src, attn_dst, Wd, bd):
    xp = jnp.concatenate(
        [node_features,
         jnp.zeros((N_PAD - N_NODES, D_FEAT), jnp.float32)], axis=0)
    ei = jnp.concatenate(
        [edge_index,
         jnp.full((2, E_PAD - N_EDGES), N_PAD - 1, jnp.int32)], axis=1)
    g, t = _prep(xp, W, attn_src, attn_dst)
    p = _edges(g, t, ei)
    return _final(p, Wd, bd.reshape(1, 1))


# speedup vs baseline: 1.0242x; 1.0242x over previous
import functools

import jax
import jax.numpy as jnp
from jax import lax
from jax.experimental import pallas as pl
from jax.experimental.pallas import tpu as pltpu
from jax.experimental.pallas import tpu_sc as plsc

N_NODES = 10000
N_EDGES = 320000
D_FEAT = 128
HIDDEN = 128
HEADS = 8
HEAD_DIM = 16
GCOLS = 144
TCOLS = 16

PREP_BLK = 1024
FINAL_BLK = 1000

NC = 2
NS = 16
CHUNK = 112
N_PAD = 10240
ROWS_PER_TILE = N_PAD // NS
NW = NC * NS
PER_TILE = 90
NPAIRS = PER_TILE // 2
NCHUNKS = PER_TILE * NW
E_PAD = NCHUNKS * CHUNK


def _prep_body(x_ref, w_ref, asrc_ref, adst_ref, g_ref, t_ref):
    x = x_ref[...]
    w = w_ref[...]
    a_s = asrc_ref[...]
    a_d = adst_ref[...]
    k = lax.broadcasted_iota(jnp.int32, (HEADS, HIDDEN), 0)
    m = lax.broadcasted_iota(jnp.int32, (HEADS, HIDDEN), 1)
    sel = (m // HEAD_DIM) == k
    Bs = jnp.where(sel, jnp.concatenate([a_s] * HEADS, axis=1), 0.0)
    Bd = jnp.where(sel, jnp.concatenate([a_d] * HEADS, axis=1), 0.0)
    h = jnp.dot(x, w, preferred_element_type=jnp.float32)
    dn = (((1,), (1,)), ((), ()))
    a_src = lax.dot_general(h, Bs, dn, preferred_element_type=jnp.float32)
    a_dst = lax.dot_general(h, Bd, dn, preferred_element_type=jnp.float32)
    zero8 = jnp.zeros_like(a_src)
    g_ref[...] = jnp.concatenate([h, a_src, zero8], axis=1)
    t_ref[...] = jnp.concatenate([a_dst, zero8], axis=1)


def _prep(x, w, attn_src, attn_dst):
    grid = N_PAD // PREP_BLK
    return pl.pallas_call(
        _prep_body,
        grid=(grid,),
        in_specs=[
            pl.BlockSpec((PREP_BLK, D_FEAT), lambda i: (i, 0)),
            pl.BlockSpec((D_FEAT, HIDDEN), lambda i: (0, 0)),
            pl.BlockSpec((HEADS, HEAD_DIM), lambda i: (0, 0)),
            pl.BlockSpec((HEADS, HEAD_DIM), lambda i: (0, 0)),
        ],
        out_specs=[
            pl.BlockSpec((PREP_BLK, GCOLS), lambda i: (i, 0)),
            pl.BlockSpec((PREP_BLK, TCOLS), lambda i: (i, 0)),
        ],
        out_shape=[
            jax.ShapeDtypeStruct((N_PAD, GCOLS), jnp.float32),
            jax.ShapeDtypeStruct((N_PAD, TCOLS), jnp.float32),
        ],
    )(x, w, attn_src, attn_dst)


def _edge_body(g_hbm, t_hbm, ei_hbm, out_hbm, sidx0, didx0, sidx1, didx1,
               gbuf0, gbuf1, tbuf0, tbuf1, acc,
               sem_g0, sem_t0, sem_g1, sem_t1):
    c = lax.axis_index("c")
    s = lax.axis_index("s")
    wid = c * NS + s
    chunk0 = wid * PER_TILE
    lanes = lax.iota(jnp.int32, 16)

    zeros16 = jnp.zeros((16,), jnp.float32)

    def zrow(i, carry):
        for kk in range(GCOLS // 16):
            gbuf0[i, pl.ds(kk * 16, 16)] = zeros16
        return carry

    lax.fori_loop(0, CHUNK, zrow, 0)

    r0 = s * ROWS_PER_TILE
    nfull = ROWS_PER_TILE // CHUNK
    for j in range(nfull):
        pltpu.sync_copy(gbuf0, acc.at[pl.ds(r0 + j * CHUNK, CHUNK)])
    rem = ROWS_PER_TILE - nfull * CHUNK
    pltpu.sync_copy(gbuf0.at[pl.ds(0, rem)],
                    acc.at[pl.ds(r0 + nfull * CHUNK, rem)])
    plsc.subcore_barrier()

    sx = (sidx0, sidx1)
    dx = (didx0, didx1)
    gb = (gbuf0, gbuf1)
    tb = (tbuf0, tbuf1)
    sg = (sem_g0, sem_g1)
    st = (sem_t0, sem_t1)

    def load_idx(j, pi):
        base = (chunk0 + j) * CHUNK
        pltpu.sync_copy(ei_hbm.at[0, pl.ds(base, CHUNK)], sx[pi])
        pltpu.sync_copy(ei_hbm.at[1, pl.ds(base, CHUNK)], dx[pi])

    def issue(pi):
        pltpu.async_copy(g_hbm.at[sx[pi]], gb[pi], sg[pi])
        pltpu.async_copy(t_hbm.at[dx[pi]], tb[pi], st[pi])

    def drain(pi):
        pltpu.make_async_copy(g_hbm.at[sx[pi]], gb[pi], sg[pi]).wait()
        pltpu.make_async_copy(t_hbm.at[dx[pi]], tb[pi], st[pi]).wait()

    def process(pi):
        gbuf, tbuf = gb[pi], tb[pi]

        def wrow(i, carry2):
            for u in range(2):
                b = i * 2 + u
                a_s = gbuf[b, pl.ds(HIDDEN, 16)]
                a_d = tbuf[b, pl.ds(0, 16)]
                e = a_s + a_d
                e = jnp.where(e >= 0.0, e, 0.2 * e)
                sv = jnp.where(lanes < HEADS, jnp.exp(e), 0.0)
                gbuf[b, pl.ds(HIDDEN, 16)] = sv
                for k in range(HEADS):
                    gbuf[b, pl.ds(k * 16, 16)] = gbuf[b, pl.ds(k * 16, 16)] * sv[k]
            return carry2

        lax.fori_loop(0, CHUNK // 2, wrow, 0)
        pltpu.sync_copy(gbuf, acc.at[dx[pi]], add=True)

    load_idx(0, 0)
    issue(0)

    def pair(jj, carry):
        j0 = 2 * jj
        load_idx(j0 + 1, 1)
        issue(1)
        drain(0)
        process(0)

        @pl.when(jj < NPAIRS - 1)
        def _():
            load_idx(j0 + 2, 0)
            issue(0)

        drain(1)
        process(1)
        return carry

    lax.fori_loop(0, NPAIRS, pair, 0)
    plsc.subcore_barrier()

    pltpu.sync_copy(acc.at[pl.ds(r0, ROWS_PER_TILE)],
                    out_hbm.at[c, pl.ds(r0, ROWS_PER_TILE)])


def _edges(g, t, edge_index2):
    mesh = plsc.VectorSubcoreMesh(core_axis_name="c", subcore_axis_name="s",
                                  num_cores=NC, num_subcores=NS)
    fn = functools.partial(
        pl.kernel,
        out_type=jax.ShapeDtypeStruct((NC, N_PAD, GCOLS), jnp.float32),
        mesh=mesh,
        scratch_types=(
            [pltpu.VMEM((CHUNK,), jnp.int32) for _ in range(4)]
            + [pltpu.VMEM((CHUNK, GCOLS), jnp.float32) for _ in range(2)]
            + [pltpu.VMEM((CHUNK, TCOLS), jnp.float32) for _ in range(2)]
            + [pltpu.VMEM_SHARED((N_PAD, GCOLS), jnp.float32)]
            + [pltpu.SemaphoreType.DMA for _ in range(4)]
        ),
        compiler_params=pltpu.CompilerParams(use_tc_tiling_on_sc=False),
    )(_edge_body)
    return fn(g, t, edge_index2)


def _final_body(p_ref, wd_ref, bd_ref, o_ref):
    ssum = p_ref[0] + p_ref[1]
    numer = ssum[:, :HIDDEN]
    denom = ssum[:, HIDDEN:HIDDEN + HEADS]
    k = lax.broadcasted_iota(jnp.int32, (HEADS, HIDDEN), 0)
    j = lax.broadcasted_iota(jnp.int32, (HEADS, HIDDEN), 1)
    rep = jnp.where((j // HEAD_DIM) == k, 1.0, 0.0)
    dex = jnp.dot(denom, rep, preferred_element_type=jnp.float32)
    x = jnp.maximum(numer / (dex + 1e-9), 0.0)
    y = jnp.dot(x, wd_ref[...], preferred_element_type=jnp.float32) + bd_ref[0, 0]
    o_ref[...] = 1.0 / (1.0 + jnp.exp(-y))


def _final(p, wd, bd):
    grid = N_NODES // FINAL_BLK
    return pl.pallas_call(
        _final_body,
        grid=(grid,),
        in_specs=[
            pl.BlockSpec((NC, FINAL_BLK, GCOLS), lambda i: (0, i, 0)),
            pl.BlockSpec((HIDDEN, 1), lambda i: (0, 0)),
            pl.BlockSpec((1, 1), lambda i: (0, 0)),
        ],
        out_specs=pl.BlockSpec((FINAL_BLK, 1), lambda i: (i, 0)),
        out_shape=jax.ShapeDtypeStruct((N_NODES, 1), jnp.float32),
    )(p, wd, bd)


@jax.jit
def kernel(node_features, edge_index, W, attn_src, attn_dst, Wd, bd):
    xp = jnp.concatenate(
        [node_features,
         jnp.zeros((N_PAD - N_NODES, D_FEAT), jnp.float32)], axis=0)
    ei = jnp.concatenate(
        [edge_index,
         jnp.full((2, E_PAD - N_EDGES), N_PAD - 1, jnp.int32)], axis=1)
    g, t = _prep(xp, W, attn_src, attn_dst)
    p = _edges(g, t, ei)
    return _final(p, Wd, bd.reshape(1, 1))


# speedup vs baseline: 1.0290x; 1.0047x over previous
import functools

import jax
import jax.numpy as jnp
from jax import lax
from jax.experimental import pallas as pl
from jax.experimental.pallas import tpu as pltpu
from jax.experimental.pallas import tpu_sc as plsc

N_NODES = 10000
N_EDGES = 320000
D_FEAT = 128
HIDDEN = 128
HEADS = 8
HEAD_DIM = 16
GCOLS = 144
TCOLS = 16

PREP_BLK = 1024
FINAL_BLK = 1000

NC = 2
NS = 16
CHUNK = 112
N_PAD = 10240
ROWS_PER_TILE = N_PAD // NS
NW = NC * NS
PER_TILE = 90
NPAIRS = PER_TILE // 2
NCHUNKS = PER_TILE * NW
E_PAD = NCHUNKS * CHUNK


def _prep_body(x_ref, w_ref, asrc_ref, adst_ref, g_ref, t_ref):
    x = x_ref[...]
    w = w_ref[...]
    a_s = asrc_ref[...]
    a_d = adst_ref[...]
    k = lax.broadcasted_iota(jnp.int32, (HEADS, HIDDEN), 0)
    m = lax.broadcasted_iota(jnp.int32, (HEADS, HIDDEN), 1)
    sel = (m // HEAD_DIM) == k
    Bs = jnp.where(sel, jnp.concatenate([a_s] * HEADS, axis=1), 0.0)
    Bd = jnp.where(sel, jnp.concatenate([a_d] * HEADS, axis=1), 0.0)
    h = jnp.dot(x, w, preferred_element_type=jnp.float32)
    dn = (((1,), (1,)), ((), ()))
    a_src = lax.dot_general(h, Bs, dn, preferred_element_type=jnp.float32)
    a_dst = lax.dot_general(h, Bd, dn, preferred_element_type=jnp.float32)
    zero8 = jnp.zeros_like(a_src)
    g_ref[...] = jnp.concatenate([h, a_src, zero8], axis=1)
    t_ref[...] = jnp.concatenate([a_dst, zero8], axis=1)


def _prep(x, w, attn_src, attn_dst):
    grid = N_PAD // PREP_BLK
    return pl.pallas_call(
        _prep_body,
        grid=(grid,),
        in_specs=[
            pl.BlockSpec((PREP_BLK, D_FEAT), lambda i: (i, 0)),
            pl.BlockSpec((D_FEAT, HIDDEN), lambda i: (0, 0)),
            pl.BlockSpec((HEADS, HEAD_DIM), lambda i: (0, 0)),
            pl.BlockSpec((HEADS, HEAD_DIM), lambda i: (0, 0)),
        ],
        out_specs=[
            pl.BlockSpec((PREP_BLK, GCOLS), lambda i: (i, 0)),
            pl.BlockSpec((PREP_BLK, TCOLS), lambda i: (i, 0)),
        ],
        out_shape=[
            jax.ShapeDtypeStruct((N_PAD, GCOLS), jnp.float32),
            jax.ShapeDtypeStruct((N_PAD, TCOLS), jnp.float32),
        ],
    )(x, w, attn_src, attn_dst)


def _edge_body(g_hbm, t_hbm, ei_hbm, out_hbm, sidx0, didx0, sidx1, didx1,
               gbuf0, gbuf1, tbuf0, tbuf1, acc,
               sem_g0, sem_t0, sem_g1, sem_t1):
    c = lax.axis_index("c")
    s = lax.axis_index("s")
    wid = c * NS + s
    chunk0 = wid * PER_TILE
    lanes = lax.iota(jnp.int32, 16)

    zeros16 = jnp.zeros((16,), jnp.float32)

    def zrow(i, carry):
        for kk in range(GCOLS // 16):
            gbuf0[i, pl.ds(kk * 16, 16)] = zeros16
        return carry

    lax.fori_loop(0, CHUNK, zrow, 0)

    r0 = s * ROWS_PER_TILE
    nfull = ROWS_PER_TILE // CHUNK
    for j in range(nfull):
        pltpu.sync_copy(gbuf0, acc.at[pl.ds(r0 + j * CHUNK, CHUNK)])
    rem = ROWS_PER_TILE - nfull * CHUNK
    pltpu.sync_copy(gbuf0.at[pl.ds(0, rem)],
                    acc.at[pl.ds(r0 + nfull * CHUNK, rem)])
    plsc.subcore_barrier()

    sx = (sidx0, sidx1)
    dx = (didx0, didx1)
    gb = (gbuf0, gbuf1)
    tb = (tbuf0, tbuf1)
    sg = (sem_g0, sem_g1)
    st = (sem_t0, sem_t1)

    def load_idx(j, pi):
        base = (chunk0 + j) * CHUNK
        pltpu.sync_copy(ei_hbm.at[0, pl.ds(base, CHUNK)], sx[pi])
        pltpu.sync_copy(ei_hbm.at[1, pl.ds(base, CHUNK)], dx[pi])

    def issue(pi):
        pltpu.async_copy(g_hbm.at[sx[pi]], gb[pi], sg[pi])
        pltpu.async_copy(t_hbm.at[dx[pi]], tb[pi], st[pi])

    def drain(pi):
        pltpu.make_async_copy(g_hbm.at[sx[pi]], gb[pi], sg[pi]).wait()
        pltpu.make_async_copy(t_hbm.at[dx[pi]], tb[pi], st[pi]).wait()

    def process(pi):
        gbuf, tbuf = gb[pi], tb[pi]

        def wrow(b, carry2):
            a_s = gbuf[b, pl.ds(HIDDEN, 16)]
            a_d = tbuf[b, pl.ds(0, 16)]
            e = a_s + a_d
            e = jnp.where(e >= 0.0, e, 0.2 * e)
            sv = jnp.where(lanes < HEADS, jnp.exp(e), 0.0)
            gbuf[b, pl.ds(HIDDEN, 16)] = sv
            for k in range(HEADS):
                gbuf[b, pl.ds(k * 16, 16)] = gbuf[b, pl.ds(k * 16, 16)] * sv[k]
            return carry2

        lax.fori_loop(0, CHUNK, wrow, 0)
        pltpu.sync_copy(gbuf, acc.at[dx[pi]], add=True)

    load_idx(0, 0)
    issue(0)

    def pair(jj, carry):
        j0 = 2 * jj
        load_idx(j0 + 1, 1)
        issue(1)
        drain(0)
        process(0)

        @pl.when(jj < NPAIRS - 1)
        def _():
            load_idx(j0 + 2, 0)
            issue(0)

        drain(1)
        process(1)
        return carry

    lax.fori_loop(0, NPAIRS, pair, 0)
    plsc.subcore_barrier()

    pltpu.sync_copy(acc.at[pl.ds(r0, ROWS_PER_TILE)],
                    out_hbm.at[c, pl.ds(r0, ROWS_PER_TILE)])


def _edges(g, t, edge_index2):
    mesh = plsc.VectorSubcoreMesh(core_axis_name="c", subcore_axis_name="s",
                                  num_cores=NC, num_subcores=NS)
    fn = functools.partial(
        pl.kernel,
        out_type=jax.ShapeDtypeStruct((NC, N_PAD, GCOLS), jnp.float32),
        mesh=mesh,
        scratch_types=(
            [pltpu.VMEM((CHUNK,), jnp.int32) for _ in range(4)]
            + [pltpu.VMEM((CHUNK, GCOLS), jnp.float32) for _ in range(2)]
            + [pltpu.VMEM((CHUNK, TCOLS), jnp.float32) for _ in range(2)]
            + [pltpu.VMEM_SHARED((N_PAD, GCOLS), jnp.float32)]
            + [pltpu.SemaphoreType.DMA for _ in range(4)]
        ),
        compiler_params=pltpu.CompilerParams(use_tc_tiling_on_sc=False),
    )(_edge_body)
    return fn(g, t, edge_index2)


def _final_body(p_ref, wd_ref, bd_ref, o_ref):
    ssum = p_ref[0] + p_ref[1]
    numer = ssum[:, :HIDDEN]
    denom = ssum[:, HIDDEN:HIDDEN + HEADS]
    k = lax.broadcasted_iota(jnp.int32, (HEADS, HIDDEN), 0)
    j = lax.broadcasted_iota(jnp.int32, (HEADS, HIDDEN), 1)
    rep = jnp.where((j // HEAD_DIM) == k, 1.0, 0.0)
    dex = jnp.dot(denom, rep, preferred_element_type=jnp.float32)
    x = jnp.maximum(numer / (dex + 1e-9), 0.0)
    y = jnp.dot(x, wd_ref[...], preferred_element_type=jnp.float32) + bd_ref[0, 0]
    o_ref[...] = 1.0 / (1.0 + jnp.exp(-y))


def _final(p, wd, bd):
    grid = N_NODES // FINAL_BLK
    return pl.pallas_call(
        _final_body,
        grid=(grid,),
        in_specs=[
            pl.BlockSpec((NC, FINAL_BLK, GCOLS), lambda i: (0, i, 0)),
            pl.BlockSpec((HIDDEN, 1), lambda i: (0, 0)),
            pl.BlockSpec((1, 1), lambda i: (0, 0)),
        ],
        out_specs=pl.BlockSpec((FINAL_BLK, 1), lambda i: (i, 0)),
        out_shape=jax.ShapeDtypeStruct((N_NODES, 1), jnp.float32),
    )(p, wd, bd)


@jax.jit
def kernel(node_features, edge_index, W, attn_src, attn_dst, Wd, bd):
    xp = jnp.concatenate(
        [node_features,
         jnp.zeros((N_PAD - N_NODES, D_FEAT), jnp.float32)], axis=0)
    ei = jnp.concatenate(
        [edge_index,
         jnp.full((2, E_PAD - N_EDGES), N_PAD - 1, jnp.int32)], axis=1)
    g, t = _prep(xp, W, attn_src, attn_dst)
    p = _edges(g, t, ei)
    return _final(p, Wd, bd.reshape(1, 1))


# speedup vs baseline: 1.0385x; 1.0092x over previous
import functools

import jax
import jax.numpy as jnp
from jax import lax
from jax.experimental import pallas as pl
from jax.experimental.pallas import tpu as pltpu
from jax.experimental.pallas import tpu_sc as plsc

N_NODES = 10000
N_EDGES = 320000
D_FEAT = 128
HIDDEN = 128
HEADS = 8
HEAD_DIM = 16
GCOLS = 144
TCOLS = 16

PREP_BLK = 1024
FINAL_BLK = 1000

NC = 2
NS = 16
CHUNK = 120
N_PAD = 10240
ROWS_PER_TILE = N_PAD // NS
NW = NC * NS
PER_TILE = 84
NPAIRS = PER_TILE // 2
NCHUNKS = PER_TILE * NW
E_PAD = NCHUNKS * CHUNK


def _prep_body(x_ref, w_ref, asrc_ref, adst_ref, g_ref, t_ref):
    x = x_ref[...]
    w = w_ref[...]
    a_s = asrc_ref[...]
    a_d = adst_ref[...]
    k = lax.broadcasted_iota(jnp.int32, (HEADS, HIDDEN), 0)
    m = lax.broadcasted_iota(jnp.int32, (HEADS, HIDDEN), 1)
    sel = (m // HEAD_DIM) == k
    Bs = jnp.where(sel, jnp.concatenate([a_s] * HEADS, axis=1), 0.0)
    Bd = jnp.where(sel, jnp.concatenate([a_d] * HEADS, axis=1), 0.0)
    h = jnp.dot(x, w, preferred_element_type=jnp.float32)
    dn = (((1,), (1,)), ((), ()))
    a_src = lax.dot_general(h, Bs, dn, preferred_element_type=jnp.float32)
    a_dst = lax.dot_general(h, Bd, dn, preferred_element_type=jnp.float32)
    zero8 = jnp.zeros_like(a_src)
    g_ref[...] = jnp.concatenate([h, a_src, zero8], axis=1)
    t_ref[...] = jnp.concatenate([a_dst, zero8], axis=1)


def _prep(x, w, attn_src, attn_dst):
    grid = N_PAD // PREP_BLK
    return pl.pallas_call(
        _prep_body,
        grid=(grid,),
        in_specs=[
            pl.BlockSpec((PREP_BLK, D_FEAT), lambda i: (i, 0)),
            pl.BlockSpec((D_FEAT, HIDDEN), lambda i: (0, 0)),
            pl.BlockSpec((HEADS, HEAD_DIM), lambda i: (0, 0)),
            pl.BlockSpec((HEADS, HEAD_DIM), lambda i: (0, 0)),
        ],
        out_specs=[
            pl.BlockSpec((PREP_BLK, GCOLS), lambda i: (i, 0)),
            pl.BlockSpec((PREP_BLK, TCOLS), lambda i: (i, 0)),
        ],
        out_shape=[
            jax.ShapeDtypeStruct((N_PAD, GCOLS), jnp.float32),
            jax.ShapeDtypeStruct((N_PAD, TCOLS), jnp.float32),
        ],
    )(x, w, attn_src, attn_dst)


def _edge_body(g_hbm, t_hbm, ei_hbm, out_hbm, sidx0, didx0, sidx1, didx1,
               gbuf0, gbuf1, tbuf0, tbuf1, acc,
               sem_g0, sem_t0, sem_g1, sem_t1):
    c = lax.axis_index("c")
    s = lax.axis_index("s")
    wid = c * NS + s
    chunk0 = wid * PER_TILE
    lanes = lax.iota(jnp.int32, 16)

    zeros16 = jnp.zeros((16,), jnp.float32)

    def zrow(i, carry):
        for kk in range(GCOLS // 16):
            gbuf0[i, pl.ds(kk * 16, 16)] = zeros16
        return carry

    lax.fori_loop(0, CHUNK, zrow, 0)

    r0 = s * ROWS_PER_TILE
    nfull = ROWS_PER_TILE // CHUNK
    for j in range(nfull):
        pltpu.sync_copy(gbuf0, acc.at[pl.ds(r0 + j * CHUNK, CHUNK)])
    rem = ROWS_PER_TILE - nfull * CHUNK
    pltpu.sync_copy(gbuf0.at[pl.ds(0, rem)],
                    acc.at[pl.ds(r0 + nfull * CHUNK, rem)])
    plsc.subcore_barrier()

    sx = (sidx0, sidx1)
    dx = (didx0, didx1)
    gb = (gbuf0, gbuf1)
    tb = (tbuf0, tbuf1)
    sg = (sem_g0, sem_g1)
    st = (sem_t0, sem_t1)

    def load_idx(j, pi):
        base = (chunk0 + j) * CHUNK
        pltpu.sync_copy(ei_hbm.at[0, pl.ds(base, CHUNK)], sx[pi])
        pltpu.sync_copy(ei_hbm.at[1, pl.ds(base, CHUNK)], dx[pi])

    def issue(pi):
        pltpu.async_copy(g_hbm.at[sx[pi]], gb[pi], sg[pi])
        pltpu.async_copy(t_hbm.at[dx[pi]], tb[pi], st[pi])

    def drain(pi):
        pltpu.make_async_copy(g_hbm.at[sx[pi]], gb[pi], sg[pi]).wait()
        pltpu.make_async_copy(t_hbm.at[dx[pi]], tb[pi], st[pi]).wait()

    def process(pi):
        gbuf, tbuf = gb[pi], tb[pi]

        def wrow(b, carry2):
            a_s = gbuf[b, pl.ds(HIDDEN, 16)]
            a_d = tbuf[b, pl.ds(0, 16)]
            e = a_s + a_d
            e = jnp.where(e >= 0.0, e, 0.2 * e)
            sv = jnp.where(lanes < HEADS, jnp.exp(e), 0.0)
            gbuf[b, pl.ds(HIDDEN, 16)] = sv
            for k in range(HEADS):
                gbuf[b, pl.ds(k * 16, 16)] = gbuf[b, pl.ds(k * 16, 16)] * sv[k]
            return carry2

        lax.fori_loop(0, CHUNK, wrow, 0)
        pltpu.sync_copy(gbuf, acc.at[dx[pi]], add=True)

    load_idx(0, 0)
    issue(0)

    def pair(jj, carry):
        j0 = 2 * jj
        load_idx(j0 + 1, 1)
        issue(1)
        drain(0)
        process(0)

        @pl.when(jj < NPAIRS - 1)
        def _():
            load_idx(j0 + 2, 0)
            issue(0)

        drain(1)
        process(1)
        return carry

    lax.fori_loop(0, NPAIRS, pair, 0)
    plsc.subcore_barrier()

    pltpu.sync_copy(acc.at[pl.ds(r0, ROWS_PER_TILE)],
                    out_hbm.at[c, pl.ds(r0, ROWS_PER_TILE)])


def _edges(g, t, edge_index2):
    mesh = plsc.VectorSubcoreMesh(core_axis_name="c", subcore_axis_name="s",
                                  num_cores=NC, num_subcores=NS)
    fn = functools.partial(
        pl.kernel,
        out_type=jax.ShapeDtypeStruct((NC, N_PAD, GCOLS), jnp.float32),
        mesh=mesh,
        scratch_types=(
            [pltpu.VMEM((CHUNK,), jnp.int32) for _ in range(4)]
            + [pltpu.VMEM((CHUNK, GCOLS), jnp.float32) for _ in range(2)]
            + [pltpu.VMEM((CHUNK, TCOLS), jnp.float32) for _ in range(2)]
            + [pltpu.VMEM_SHARED((N_PAD, GCOLS), jnp.float32)]
            + [pltpu.SemaphoreType.DMA for _ in range(4)]
        ),
        compiler_params=pltpu.CompilerParams(use_tc_tiling_on_sc=False),
    )(_edge_body)
    return fn(g, t, edge_index2)


def _final_body(p_ref, wd_ref, bd_ref, o_ref):
    ssum = p_ref[0] + p_ref[1]
    numer = ssum[:, :HIDDEN]
    denom = ssum[:, HIDDEN:HIDDEN + HEADS]
    k = lax.broadcasted_iota(jnp.int32, (HEADS, HIDDEN), 0)
    j = lax.broadcasted_iota(jnp.int32, (HEADS, HIDDEN), 1)
    rep = jnp.where((j // HEAD_DIM) == k, 1.0, 0.0)
    dex = jnp.dot(denom, rep, preferred_element_type=jnp.float32)
    x = jnp.maximum(numer / (dex + 1e-9), 0.0)
    y = jnp.dot(x, wd_ref[...], preferred_element_type=jnp.float32) + bd_ref[0, 0]
    o_ref[...] = 1.0 / (1.0 + jnp.exp(-y))


def _final(p, wd, bd):
    grid = N_NODES // FINAL_BLK
    return pl.pallas_call(
        _final_body,
        grid=(grid,),
        in_specs=[
            pl.BlockSpec((NC, FINAL_BLK, GCOLS), lambda i: (0, i, 0)),
            pl.BlockSpec((HIDDEN, 1), lambda i: (0, 0)),
            pl.BlockSpec((1, 1), lambda i: (0, 0)),
        ],
        out_specs=pl.BlockSpec((FINAL_BLK, 1), lambda i: (i, 0)),
        out_shape=jax.ShapeDtypeStruct((N_NODES, 1), jnp.float32),
    )(p, wd, bd)


@jax.jit
def kernel(node_features, edge_index, W, attn_src, attn_dst, Wd, bd):
    xp = jnp.concatenate(
        [node_features,
         jnp.zeros((N_PAD - N_NODES, D_FEAT), jnp.float32)], axis=0)
    ei = jnp.concatenate(
        [edge_index,
         jnp.full((2, E_PAD - N_EDGES), N_PAD - 1, jnp.int32)], axis=1)
    g, t = _prep(xp, W, attn_src, attn_dst)
    p = _edges(g, t, ei)
    return _final(p, Wd, bd.reshape(1, 1))
